# baseline (speedup 1.0000x reference)
import functools

import jax
import jax.numpy as jnp
from jax import lax
from jax.experimental import pallas as pl
from jax.experimental.pallas import tpu as pltpu
from jax.experimental.pallas import tpu_sc as plsc

BATCH = 16384
L = 16
NI = 100
NC = 2
NS = 16
NW = NC * NS
BPW = BATCH // NW
CH = 128
NCH = BPW // CH

RB = 10240
TBLK = 2048
NTB = BATCH // TBLK


def _relayout_body(*refs):
    parts = [refs[i][...] for i in range(5)]
    parts += [refs[5 + i][...] for i in range(6)]
    parts.append(jnp.zeros((128 - 5 * L - 6, RB), jnp.float32))
    x = jnp.concatenate(parts, axis=0)
    refs[11][...] = jnp.transpose(x, (1, 0)).astype(jnp.bfloat16)


def _relayout(ue_tabs, us_tabs):
    U = ue_tabs[0].shape[0]
    n = (U + RB - 1) // RB
    ins = [t.T for t in ue_tabs] + [t.T for t in us_tabs]
    in_specs = ([pl.BlockSpec((L, RB), lambda i: (0, i))] * 5
                + [pl.BlockSpec((1, RB), lambda i: (0, i))] * 6)
    out = pl.pallas_call(
        _relayout_body,
        grid=(n,),
        in_specs=in_specs,
        out_specs=pl.BlockSpec((RB, 128), lambda i: (i, 0)),
        out_shape=jax.ShapeDtypeStruct((U, 128), jnp.bfloat16),
    )(*ins)
    return out


def _sc_gather(users, table):
    mesh = plsc.VectorSubcoreMesh(core_axis_name="c", subcore_axis_name="s")
    out_type = jax.ShapeDtypeStruct((BATCH, 128), jnp.bfloat16)
    scratch = ([pltpu.VMEM((NCH, CH), jnp.int32)]
               + [pltpu.VMEM((CH, 128), jnp.bfloat16)] * NCH
               + [pltpu.SemaphoreType.DMA])

    @functools.partial(pl.kernel, mesh=mesh, out_type=out_type,
                       scratch_types=scratch,
                       compiler_params=pltpu.CompilerParams(
                           use_tc_tiling_on_sc=False))
    def k(users_hbm, tab, out, idx_v, *rest):
        bufs = rest[0:NCH]
        sem = rest[NCH]
        wid = lax.axis_index("s") * NC + lax.axis_index("c")
        base = wid * BPW
        for c in range(NCH):
            pltpu.sync_copy(users_hbm.at[pl.ds(base + c * CH, CH)],
                            idx_v.at[c])
        cps = [pltpu.async_copy(tab.at[idx_v.at[c]], bufs[c], sem)
               for c in range(NCH)]
        for c in range(NCH):
            cps[c].wait()
            pltpu.sync_copy(bufs[c], out.at[pl.ds(base + c * CH, CH), :])

    return k(users, table)


def _tc_body(items_ref, rows_ref,
             dist_ref, price_ref,
             iba, ibb, ibl, ibg, ibd,
             iea, ieb, iel, ieg, ied,
             gba, gbb, gbl, gbg, gbd,
             out_ref):
    it = items_ref[...]
    rows = rows_ref[...].astype(jnp.float32)
    onehot = (it == lax.broadcasted_iota(jnp.int32, (TBLK, NI), 1)
              ).astype(jnp.float32)

    def ig(r):
        return jnp.dot(onehot, r[...], preferred_element_type=jnp.float32)

    def coef(i, gb, ib, ie):
        ue = rows[:, L * i:L * (i + 1)]
        ub = rows[:, 80 + i:81 + i]
        d = jnp.sum(ue * ig(ie), axis=1, keepdims=True)
        return gb[0, 0] + ub + ig(ib) + d

    alpha = coef(0, gba, iba, iea)
    beta = coef(1, gbb, ibb, ieb)
    lamda = coef(2, gbl, ibl, iel)
    gamma = coef(3, gbg, ibg, ieg)
    delta = coef(4, gbd, ibd, ied)
    refr = rows[:, 85:86]

    dist = ig(dist_ref)
    price = ig(price_ref)

    rating = lax.broadcasted_iota(jnp.int32, (TBLK, 5), 1).astype(jnp.float32) + 1.0
    x = jnp.tanh(rating - refr)
    x_pos = (x > 0).astype(jnp.float32)
    x_neg = 1.0 - x_pos
    x_ = price * jnp.abs(x)
    v = x_ ** (alpha * x_pos + beta * x_neg)
    value = v * (x_pos - lamda * x_neg)
    w_exp = x_pos * gamma + x_neg * delta
    w_nom = dist ** w_exp
    w_den = (w_nom + (1.0 - dist) ** w_exp) ** (1.0 / w_exp)
    out_ref[...] = jnp.sum((w_nom / w_den) * value, axis=1, keepdims=True)


def _tc_math(items, rows, p):
    full = lambda a: pl.BlockSpec(a.shape, lambda i: (0,) * a.ndim)

    items2 = items.reshape(BATCH, 1)
    dist = p["dist"]
    price2 = p["price"].reshape(NI, 1)
    ibs = [p["ib_" + t] for t in ("a", "b", "l", "g", "d")]
    ies = [p["ie_" + t] for t in ("a", "b", "l", "g", "d")]
    gbs = [p["gb_" + t] for t in ("a", "b", "l", "g", "d")]

    args = [items2, rows, dist, price2] + ibs + ies + gbs
    specs = ([pl.BlockSpec((TBLK, 1), lambda i: (i, 0)),
              pl.BlockSpec((TBLK, 128), lambda i: (i, 0)),
              full(dist), full(price2)]
             + [full(a) for a in ibs] + [full(a) for a in ies]
             + [full(a) for a in gbs])
    out = pl.pallas_call(
        _tc_body,
        grid=(NTB,),
        in_specs=specs,
        out_specs=pl.BlockSpec((TBLK, 1), lambda i: (i, 0)),
        out_shape=jax.ShapeDtypeStruct((BATCH, 1), jnp.float32),
    )(*args)
    return out.reshape(BATCH)


def kernel(params, users, items):
    p = params
    ue_tabs = [p["ue_" + t] for t in ("a", "b", "l", "g", "d")]
    us_tabs = [p["ub_" + t] for t in ("a", "b", "l", "g", "d")]
    us_tabs.append(p["ref"])
    table = _relayout(ue_tabs, us_tabs)
    rows = _sc_gather(users, table)
    return _tc_math(items, rows, p)


# speedup vs baseline: 2.8455x; 2.8455x over previous
import functools

import jax
import jax.numpy as jnp
from jax import lax
from jax.experimental import pallas as pl
from jax.experimental.pallas import tpu as pltpu
from jax.experimental.pallas import tpu_sc as plsc

BATCH = 16384
L = 16
NI = 100
NC = 2
NS = 16
NW = NC * NS
BPW = BATCH // NW
CH = 128
NCH = BPW // CH

RB = 20480
TBLK = 4096
NTB = BATCH // TBLK


def _relayout_body(*refs):
    parts = [refs[i][...] for i in range(5)]
    parts += [refs[5 + i][...] for i in range(6)]
    parts.append(jnp.zeros((128 - 5 * L - 6, RB), jnp.float32))
    x = jnp.concatenate(parts, axis=0)
    refs[11][...] = jnp.transpose(x, (1, 0))


def _relayout(ue_tabs, us_tabs):
    U = ue_tabs[0].shape[0]
    n = (U + RB - 1) // RB
    ins = [t.T for t in ue_tabs] + [t.T for t in us_tabs]
    in_specs = ([pl.BlockSpec((L, RB), lambda i: (0, i))] * 5
                + [pl.BlockSpec((1, RB), lambda i: (0, i))] * 6)
    out = pl.pallas_call(
        _relayout_body,
        grid=(n,),
        in_specs=in_specs,
        out_specs=pl.BlockSpec((RB, 128), lambda i: (i, 0)),
        out_shape=jax.ShapeDtypeStruct((U, 128), jnp.float32),
    )(*ins)
    return out


def _sc_gather(users, table):
    mesh = plsc.VectorSubcoreMesh(core_axis_name="c", subcore_axis_name="s")
    out_type = jax.ShapeDtypeStruct((BATCH, 128), jnp.float32)
    scratch = ([pltpu.VMEM((NCH, CH), jnp.int32)]
               + [pltpu.VMEM((CH, 128), jnp.float32)] * NCH
               + [pltpu.SemaphoreType.DMA])

    @functools.partial(pl.kernel, mesh=mesh, out_type=out_type,
                       scratch_types=scratch,
                       compiler_params=pltpu.CompilerParams(
                           use_tc_tiling_on_sc=False))
    def k(users_hbm, tab, out, idx_v, *rest):
        bufs = rest[0:NCH]
        sem = rest[NCH]
        wid = lax.axis_index("s") * NC + lax.axis_index("c")
        base = wid * BPW
        for c in range(NCH):
            pltpu.sync_copy(users_hbm.at[pl.ds(base + c * CH, CH)],
                            idx_v.at[c])
        cps = [pltpu.async_copy(tab.at[idx_v.at[c]], bufs[c], sem)
               for c in range(NCH)]
        for c in range(NCH):
            cps[c].wait()
            pltpu.sync_copy(bufs[c], out.at[pl.ds(base + c * CH, CH), :])

    return k(users, table)


def _tc_body(items_ref, rows_ref,
             dist_ref, price_ref,
             iba, ibb, ibl, ibg, ibd,
             iea, ieb, iel, ieg, ied,
             gba, gbb, gbl, gbg, gbd,
             out_ref):
    it = items_ref[...]
    rows = rows_ref[...].astype(jnp.float32)
    onehot = (it == lax.broadcasted_iota(jnp.int32, (TBLK, NI), 1)
              ).astype(jnp.float32)

    def ig(r):
        return jnp.dot(onehot, r[...], preferred_element_type=jnp.float32)

    def coef(i, gb, ib, ie):
        ue = rows[:, L * i:L * (i + 1)]
        ub = rows[:, 80 + i:81 + i]
        d = jnp.sum(ue * ig(ie), axis=1, keepdims=True)
        return gb[0, 0] + ub + ig(ib) + d

    alpha = coef(0, gba, iba, iea)
    beta = coef(1, gbb, ibb, ieb)
    lamda = coef(2, gbl, ibl, iel)
    gamma = coef(3, gbg, ibg, ieg)
    delta = coef(4, gbd, ibd, ied)
    refr = rows[:, 85:86]

    dist = ig(dist_ref)
    price = ig(price_ref)

    rating = lax.broadcasted_iota(jnp.int32, (TBLK, 5), 1).astype(jnp.float32) + 1.0
    x = jnp.tanh(rating - refr)
    x_pos = (x > 0).astype(jnp.float32)
    x_neg = 1.0 - x_pos
    x_ = price * jnp.abs(x)
    v = x_ ** (alpha * x_pos + beta * x_neg)
    value = v * (x_pos - lamda * x_neg)
    w_exp = x_pos * gamma + x_neg * delta
    w_nom = dist ** w_exp
    w_den = (w_nom + (1.0 - dist) ** w_exp) ** (1.0 / w_exp)
    out_ref[...] = jnp.sum((w_nom / w_den) * value, axis=1, keepdims=True)


def _tc_math(items, rows, p):
    full = lambda a: pl.BlockSpec(a.shape, lambda i: (0,) * a.ndim)

    items2 = items.reshape(BATCH, 1)
    dist = p["dist"]
    price2 = p["price"].reshape(NI, 1)
    ibs = [p["ib_" + t] for t in ("a", "b", "l", "g", "d")]
    ies = [p["ie_" + t] for t in ("a", "b", "l", "g", "d")]
    gbs = [p["gb_" + t] for t in ("a", "b", "l", "g", "d")]

    args = [items2, rows, dist, price2] + ibs + ies + gbs
    specs = ([pl.BlockSpec((TBLK, 1), lambda i: (i, 0)),
              pl.BlockSpec((TBLK, 128), lambda i: (i, 0)),
              full(dist), full(price2)]
             + [full(a) for a in ibs] + [full(a) for a in ies]
             + [full(a) for a in gbs])
    out = pl.pallas_call(
        _tc_body,
        grid=(NTB,),
        in_specs=specs,
        out_specs=pl.BlockSpec((TBLK, 1), lambda i: (i, 0)),
        out_shape=jax.ShapeDtypeStruct((BATCH, 1), jnp.float32),
    )(*args)
    return out.reshape(BATCH)


def kernel(params, users, items):
    p = params
    ue_tabs = [p["ue_" + t] for t in ("a", "b", "l", "g", "d")]
    us_tabs = [p["ub_" + t] for t in ("a", "b", "l", "g", "d")]
    us_tabs.append(p["ref"])
    table = _relayout(ue_tabs, us_tabs)
    rows = _sc_gather(users, table)
    return _tc_math(items, rows, p)


# speedup vs baseline: 3.1805x; 1.1177x over previous
import functools

import jax
import jax.numpy as jnp
from jax import lax
from jax.experimental import pallas as pl
from jax.experimental.pallas import tpu as pltpu
from jax.experimental.pallas import tpu_sc as plsc

BATCH = 16384
L = 16
NI = 100
NC = 2
NS = 16
NW = NC * NS
BPW = BATCH // NW
CH = 128
NCH = BPW // CH

RB = 20480
TBLK = 4096
NTB = BATCH // TBLK


def _relayout_body(*refs):
    parts = [refs[i][...] for i in range(5)]
    parts += [refs[5 + i][...] for i in range(6)]
    parts.append(jnp.zeros((128 - 5 * L - 6, RB), jnp.float32))
    x = jnp.concatenate(parts, axis=0)
    refs[11][...] = jnp.transpose(x, (1, 0))


def _relayout(ue_tabs, us_tabs):
    U = ue_tabs[0].shape[0]
    n = (U + RB - 1) // RB
    ins = [t.T for t in ue_tabs] + [t.T for t in us_tabs]
    in_specs = ([pl.BlockSpec((L, RB), lambda i: (0, i))] * 5
                + [pl.BlockSpec((1, RB), lambda i: (0, i))] * 6)
    out = pl.pallas_call(
        _relayout_body,
        grid=(n,),
        in_specs=in_specs,
        out_specs=pl.BlockSpec((RB, 128), lambda i: (i, 0)),
        out_shape=jax.ShapeDtypeStruct((U, 128), jnp.float32),
    )(*ins)
    return out


def _sc_gather(users, table):
    mesh = plsc.VectorSubcoreMesh(core_axis_name="c", subcore_axis_name="s")
    out_type = jax.ShapeDtypeStruct((BATCH, 128), jnp.float32)
    scratch = ([pltpu.VMEM((NCH, CH), jnp.int32)]
               + [pltpu.VMEM((CH, 128), jnp.float32)] * NCH
               + [pltpu.SemaphoreType.DMA])

    @functools.partial(pl.kernel, mesh=mesh, out_type=out_type,
                       scratch_types=scratch,
                       compiler_params=pltpu.CompilerParams(
                           use_tc_tiling_on_sc=False))
    def k(users_hbm, tab, out, idx_v, *rest):
        bufs = rest[0:NCH]
        sem = rest[NCH]
        wid = lax.axis_index("s") * NC + lax.axis_index("c")
        base = wid * BPW
        for c in range(NCH):
            pltpu.sync_copy(users_hbm.at[pl.ds(base + c * CH, CH)],
                            idx_v.at[c])
        cps = [pltpu.async_copy(tab.at[idx_v.at[c]], bufs[c], sem)
               for c in range(NCH)]
        for c in range(NCH):
            cps[c].wait()
            pltpu.sync_copy(bufs[c], out.at[pl.ds(base + c * CH, CH), :])

    return k(users, table)


def _tc_body(items_ref, rows_ref,
             dist_ref, price_ref,
             iba, ibb, ibl, ibg, ibd,
             iea, ieb, iel, ieg, ied,
             gba, gbb, gbl, gbg, gbd,
             out_ref):
    it = items_ref[...]
    rows = rows_ref[...].astype(jnp.float32)
    onehot = (it == lax.broadcasted_iota(jnp.int32, (TBLK, NI), 1)
              ).astype(jnp.float32)

    def ig(r):
        return jnp.dot(onehot, r[...], preferred_element_type=jnp.float32)

    def coef(i, gb, ib, ie):
        ue = rows[:, L * i:L * (i + 1)]
        ub = rows[:, 80 + i:81 + i]
        d = jnp.sum(ue * ig(ie), axis=1, keepdims=True)
        return gb[0, 0] + ub + ig(ib) + d

    alpha = coef(0, gba, iba, iea)
    beta = coef(1, gbb, ibb, ieb)
    lamda = coef(2, gbl, ibl, iel)
    gamma = coef(3, gbg, ibg, ieg)
    delta = coef(4, gbd, ibd, ied)
    refr = rows[:, 85:86]

    dvals = dist_ref[...]
    ldist_g = ig(jnp.log(dvals))
    l1mdist_g = ig(jnp.log(1.0 - dvals))
    lprice = ig(jnp.log(price_ref[...]))

    rating = lax.broadcasted_iota(jnp.int32, (TBLK, 5), 1).astype(jnp.float32) + 1.0
    x = jnp.tanh(rating - refr)
    x_pos = (x > 0).astype(jnp.float32)
    x_neg = 1.0 - x_pos
    lx_ = lprice + jnp.log(jnp.abs(x))
    v = jnp.exp((alpha * x_pos + beta * x_neg) * lx_)
    value = v * (x_pos - lamda * x_neg)
    w_exp = x_pos * gamma + x_neg * delta
    w_nom = jnp.exp(w_exp * ldist_g)
    w_1m = jnp.exp(w_exp * l1mdist_g)
    w_den = jnp.exp(jnp.log(w_nom + w_1m) / w_exp)
    out_ref[...] = jnp.sum((w_nom / w_den) * value, axis=1, keepdims=True)


def _tc_math(items, rows, p):
    full = lambda a: pl.BlockSpec(a.shape, lambda i: (0,) * a.ndim)

    items2 = items.reshape(BATCH, 1)
    dist = p["dist"]
    price2 = p["price"].reshape(NI, 1)
    ibs = [p["ib_" + t] for t in ("a", "b", "l", "g", "d")]
    ies = [p["ie_" + t] for t in ("a", "b", "l", "g", "d")]
    gbs = [p["gb_" + t] for t in ("a", "b", "l", "g", "d")]

    args = [items2, rows, dist, price2] + ibs + ies + gbs
    specs = ([pl.BlockSpec((TBLK, 1), lambda i: (i, 0)),
              pl.BlockSpec((TBLK, 128), lambda i: (i, 0)),
              full(dist), full(price2)]
             + [full(a) for a in ibs] + [full(a) for a in ies]
             + [full(a) for a in gbs])
    out = pl.pallas_call(
        _tc_body,
        grid=(NTB,),
        in_specs=specs,
        out_specs=pl.BlockSpec((TBLK, 1), lambda i: (i, 0)),
        out_shape=jax.ShapeDtypeStruct((BATCH, 1), jnp.float32),
    )(*args)
    return out.reshape(BATCH)


def kernel(params, users, items):
    p = params
    ue_tabs = [p["ue_" + t] for t in ("a", "b", "l", "g", "d")]
    us_tabs = [p["ub_" + t] for t in ("a", "b", "l", "g", "d")]
    us_tabs.append(p["ref"])
    table = _relayout(ue_tabs, us_tabs)
    rows = _sc_gather(users, table)
    return _tc_math(items, rows, p)
